# baseline (speedup 1.0000x reference)
import functools

import jax
import jax.numpy as jnp
from jax import lax
from jax.experimental import pallas as pl
from jax.experimental.pallas import tpu as pltpu
from jax.experimental.pallas import tpu_sc as plsc

_B, _N, _F, _C, _K = 2, 4096, 128, 32, 16
_PW = 16
_TW = 128

_TN1 = 128
_NB1 = _N // _TN1
_TN2 = 512
_NB2 = _N // _TN2

_TOTB = _K * _N
_NWORK = 32
_CHUNK = 128
_PERW = _TOTB // _NWORK
_NCH = _PERW // _CHUNK


def _proj_topk_body(feat_ref, posb_ref, post_ref, wenc_ref, benc_ref,
                    wkvq_ref, bkvq_ref, q_ref, tab_ref, idx_ref):
    feat = feat_ref[0]
    posb = posb_ref[0]
    post = post_ref[0]

    enc = jnp.dot(feat, wenc_ref[...], preferred_element_type=jnp.float32) + benc_ref[...]
    kvq = jnp.dot(enc, wkvq_ref[...], preferred_element_type=jnp.float32) + bkvq_ref[...]
    q_ref[0] = kvq[:, 2 * _C:]
    tab_ref[0] = jnp.concatenate(
        [kvq[:, :2 * _C], posb, jnp.zeros((_TN1, _TW - 2 * _C - 8), jnp.float32)], axis=1)

    sq_all = jnp.sum(post * post, axis=0, keepdims=True)
    sq_blk = jnp.sum(posb * posb, axis=1, keepdims=True)
    dot = jax.lax.dot_general(posb, post, (((1,), (0,)), ((), ())),
                              preferred_element_type=jnp.float32)
    d = sq_blk + sq_all - 2.0 * dot

    NG = 256
    GK = _N // NG
    iota_ng = lax.broadcasted_iota(jnp.int32, (_TN1, NG), 1)
    inf = jnp.float32(jnp.inf)

    mv1 = d[:, 0:NG]
    mi1 = jnp.zeros((_TN1, NG), jnp.float32)
    mv2 = jnp.full((_TN1, NG), inf)
    mi2 = jnp.zeros((_TN1, NG), jnp.float32)
    mv3 = jnp.full((_TN1, NG), inf)
    mi3 = jnp.zeros((_TN1, NG), jnp.float32)
    for k in range(1, GK):
        v = d[:, k * NG:(k + 1) * NG]
        kf_ = jnp.float32(k)
        c1 = v < mv1
        c2 = v < mv2
        c3 = v < mv3
        mv3 = jnp.where(c2, mv2, jnp.where(c3, v, mv3))
        mi3 = jnp.where(c2, mi2, jnp.where(c3, kf_, mi3))
        mv2 = jnp.where(c1, mv1, jnp.where(c2, v, mv2))
        mi2 = jnp.where(c1, mi1, jnp.where(c2, kf_, mi2))
        mv1 = jnp.where(c1, v, mv1)
        mi1 = jnp.where(c1, kf_, mi1)

    iota_f = iota_ng.astype(jnp.float32)
    ngf = jnp.float32(NG)
    ci1 = mi1 * ngf + iota_f
    ci2 = mi2 * ngf + iota_f
    ci3 = mi3 * ngf + iota_f
    bigf = jnp.float32(1e9)
    for k in range(_K):
        m = jnp.min(mv1, axis=1, keepdims=True)
        cand = jnp.where(mv1 == m, ci1, bigf)
        am = jnp.min(cand, axis=1, keepdims=True)
        idx_ref[0, k, :] = am.astype(jnp.int32)[:, 0]
        hit = ci1 == am
        mv1 = jnp.where(hit, mv2, mv1)
        ci1 = jnp.where(hit, ci2, ci1)
        mv2 = jnp.where(hit, mv3, mv2)
        ci2 = jnp.where(hit, ci3, ci2)
        mv3 = jnp.where(hit, inf, mv3)


def _proj_topk(b, feat, posb, post, wenc, benc, wkvq, bkvq):
    return pl.pallas_call(
        _proj_topk_body,
        grid=(_NB1,),
        in_specs=[
            pl.BlockSpec((1, _TN1, _F), lambda i: (b, i, 0)),
            pl.BlockSpec((1, _TN1, 8), lambda i: (b, i, 0)),
            pl.BlockSpec((1, 8, _N), lambda i: (b, 0, 0)),
            pl.BlockSpec((_F, _C), lambda i: (0, 0)),
            pl.BlockSpec((1, _C), lambda i: (0, 0)),
            pl.BlockSpec((_C, 3 * _C), lambda i: (0, 0)),
            pl.BlockSpec((1, 3 * _C), lambda i: (0, 0)),
        ],
        out_specs=[
            pl.BlockSpec((1, _TN1, _C), lambda i: (0, i, 0)),
            pl.BlockSpec((1, _TN1, _TW), lambda i: (0, i, 0)),
            pl.BlockSpec((1, _K, _TN1), lambda i: (0, 0, i)),
        ],
        out_shape=[
            jax.ShapeDtypeStruct((1, _N, _C), jnp.float32),
            jax.ShapeDtypeStruct((1, _N, _TW), jnp.float32),
            jax.ShapeDtypeStruct((1, _K, _N), jnp.int32),
        ],
    )(feat, posb, post, wenc, benc, wkvq, bkvq)


def _sc_gather_body(tab_hbm, idx_hbm, out_hbm, idx_v, rows_v,
                    gsem0, gsem1, osem0, osem1):
    wid = lax.axis_index("s") * 2 + lax.axis_index("c")
    base = wid * _PERW
    gsem = (gsem0, gsem1)
    osem = (osem0, osem1)
    pltpu.sync_copy(idx_hbm.at[pl.ds(base, _PERW)], idx_v)

    def body(g, carry):
        @pl.when(g > 0)
        def _():
            for t in range(2):
                pltpu.make_async_copy(
                    rows_v.at[t], out_hbm.at[pl.ds(base, _CHUNK)],
                    osem[t]).wait()
        cps = []
        for t in range(2):
            c = g * 2 + t
            cps.append(pltpu.async_copy(
                tab_hbm.at[idx_v.at[pl.ds(c * _CHUNK, _CHUNK)]],
                rows_v.at[t], gsem[t]))
        for t in range(2):
            c = g * 2 + t
            cps[t].wait()
            pltpu.async_copy(
                rows_v.at[t], out_hbm.at[pl.ds(base + c * _CHUNK, _CHUNK)],
                osem[t])
        return carry

    lax.fori_loop(0, _NCH // 2, body, 0)
    for t in range(2):
        pltpu.make_async_copy(
            rows_v.at[t], out_hbm.at[pl.ds(base, _CHUNK)], osem[t]).wait()


def _sc_gather(tabf, idxf):
    mesh = plsc.VectorSubcoreMesh(core_axis_name="c", subcore_axis_name="s")
    fn = functools.partial(
        pl.kernel,
        mesh=mesh,
        out_type=jax.ShapeDtypeStruct((_TOTB, _TW), jnp.float32),
        scratch_types=[
            pltpu.VMEM((_PERW,), jnp.int32),
            pltpu.VMEM((2, _CHUNK, _TW), jnp.float32),
            pltpu.SemaphoreType.DMA,
            pltpu.SemaphoreType.DMA,
            pltpu.SemaphoreType.DMA,
            pltpu.SemaphoreType.DMA,
        ],
    )(_sc_gather_body)
    return fn(tabf, idxf)


def _attn_body(g_ref, feat_ref, q_ref, posb_ref,
               wp1_ref, bp1_ref, wp2_ref, bp2_ref,
               wg1_ref, bg1_ref, wg2_ref, bg2_ref,
               wdec_ref, bdec_ref, out_ref):
    g = g_ref[...]
    kt = _K * _TN2
    kn = g[:, :, :_C].reshape(kt, _C)
    vn = g[:, :, _C:2 * _C]
    pn = g[:, :, 2 * _C:2 * _C + _PW].reshape(kt, _PW)

    posb = posb_ref[0]
    pos16 = jnp.concatenate(
        [posb, jnp.zeros((_TN2, _PW - 8), jnp.float32)], axis=1)
    rel = jnp.broadcast_to(pos16[None], (_K, _TN2, _PW)).reshape(kt, _PW) - pn

    h = jnp.maximum(
        jnp.dot(rel, wp1_ref[...], preferred_element_type=jnp.float32) + bp1_ref[...], 0.0)
    delta = jnp.dot(h, wp2_ref[...], preferred_element_type=jnp.float32) + bp2_ref[...]

    qb = q_ref[0]
    qq = jnp.broadcast_to(qb[None], (_K, _TN2, _C)).reshape(kt, _C)
    x = qq - kn + delta
    hg = jnp.maximum(
        jnp.dot(x, wg1_ref[...], preferred_element_type=jnp.float32) + bg1_ref[...], 0.0)
    gam = jnp.dot(hg, wg2_ref[...], preferred_element_type=jnp.float32) + bg2_ref[...]

    gam3 = gam.reshape(_K, _TN2, _C)
    del3 = delta.reshape(_K, _TN2, _C)
    m = gam3[0]
    for k in range(1, _K):
        m = jnp.maximum(m, gam3[k])
    s = jnp.zeros((_TN2, _C), jnp.float32)
    acc = jnp.zeros((_TN2, _C), jnp.float32)
    for k in range(_K):
        e = jnp.exp(gam3[k] - m)
        s = s + e
        acc = acc + e * (vn[k] + del3[k])
    agg = acc / s

    out_ref[0] = feat_ref[0] + (
        jnp.dot(agg, wdec_ref[...], preferred_element_type=jnp.float32) + bdec_ref[...])


def _attn(b, g3, feat, q, posb, wp1p, bp1, wp2, bp2, wg1, bg1, wg2, bg2,
          wdec, bdec):
    wspec = pl.BlockSpec((_C, _C), lambda i: (0, 0))
    bspec = pl.BlockSpec((1, _C), lambda i: (0, 0))
    return pl.pallas_call(
        _attn_body,
        grid=(_NB2,),
        in_specs=[
            pl.BlockSpec((_K, _TN2, _TW), lambda i: (0, i, 0)),
            pl.BlockSpec((1, _TN2, _F), lambda i: (b, i, 0)),
            pl.BlockSpec((1, _TN2, _C), lambda i: (0, i, 0)),
            pl.BlockSpec((1, _TN2, 8), lambda i: (b, i, 0)),
            pl.BlockSpec((_PW, _C), lambda i: (0, 0)),
            bspec, wspec, bspec, wspec, bspec, wspec, bspec,
            pl.BlockSpec((_C, _F), lambda i: (0, 0)),
            pl.BlockSpec((1, _F), lambda i: (0, 0)),
        ],
        out_specs=pl.BlockSpec((1, _TN2, _F), lambda i: (0, i, 0)),
        out_shape=jax.ShapeDtypeStruct((1, _N, _F), jnp.float32),
    )(g3, feat, q, posb, wp1p, bp1, wp2, bp2, wg1, bg1, wg2, bg2, wdec, bdec)


def kernel(features, positions, batch, W_enc, b_enc, W_q, b_q, W_k, b_k,
           W_v, b_v, W_p1, b_p1, W_p2, b_p2, W_g1, b_g1, W_g2, b_g2,
           W_dec, b_dec):
    posb = jnp.concatenate(
        [positions, jnp.zeros((_B, _N, 5), jnp.float32)], axis=2)
    post = jnp.transpose(posb, (0, 2, 1))
    wp1p = jnp.concatenate(
        [W_p1, jnp.zeros((_PW - 3, _C), jnp.float32)], axis=0)
    wkvq = jnp.concatenate([W_k, W_v, W_q], axis=1)
    bkvq = jnp.concatenate([b_k, b_v, b_q]).reshape(1, 3 * _C)

    outs = []
    for b in range(_B):
        q, tab, idx = _proj_topk(
            b, features, posb, post, W_enc, b_enc.reshape(1, _C), wkvq, bkvq)
        gat = _sc_gather(tab.reshape(_N, _TW), idx.reshape(_TOTB))
        g3 = gat.reshape(_K, _N, _TW)
        outs.append(_attn(
            b, g3, features, q, posb, wp1p,
            b_p1.reshape(1, _C), W_p2, b_p2.reshape(1, _C),
            W_g1, b_g1.reshape(1, _C), W_g2, b_g2.reshape(1, _C),
            W_dec, b_dec.reshape(1, _F)))
    out = jnp.concatenate(outs, axis=0)
    return out, positions, batch


# speedup vs baseline: 1.0707x; 1.0707x over previous
import functools

import jax
import jax.numpy as jnp
from jax import lax
from jax.experimental import pallas as pl
from jax.experimental.pallas import tpu as pltpu
from jax.experimental.pallas import tpu_sc as plsc

_B, _N, _F, _C, _K = 2, 4096, 128, 32, 16
_PW = 16
_TW = 128

_TN1 = 128
_NB1 = _N // _TN1
_TN2 = 512
_NB2 = _N // _TN2

_TOTB = _K * _N
_NWORK = 32
_CHUNK = 128
_PERW = _TOTB // _NWORK
_NCH = _PERW // _CHUNK


def _proj_topk_body(feat_ref, posb_ref, post_ref, wenc_ref, benc_ref,
                    wkvq_ref, bkvq_ref, q_ref, tab_ref, idx_ref):
    feat = feat_ref[0]
    posb = posb_ref[0]
    post = post_ref[0]

    enc = jnp.dot(feat, wenc_ref[...], preferred_element_type=jnp.float32) + benc_ref[...]
    kvq = jnp.dot(enc, wkvq_ref[...], preferred_element_type=jnp.float32) + bkvq_ref[...]
    q_ref[0] = kvq[:, 2 * _C:]
    tab_ref[0] = jnp.concatenate(
        [kvq[:, :2 * _C], posb, jnp.zeros((_TN1, _TW - 2 * _C - 8), jnp.float32)], axis=1)

    sq_all = jnp.sum(post * post, axis=0, keepdims=True)
    sq_blk = jnp.sum(posb * posb, axis=1, keepdims=True)
    dot = jax.lax.dot_general(posb, post, (((1,), (0,)), ((), ())),
                              preferred_element_type=jnp.float32)
    d = sq_blk + sq_all - 2.0 * dot

    NG = 256
    GK = _N // NG
    inf = jnp.float32(jnp.inf)
    maskk = jnp.int32(-16)

    bias = jnp.int32(1 << 23)

    def pack(v, k):
        vi = lax.bitcast_convert_type(jnp.maximum(v, 0.0), jnp.int32)
        return lax.bitcast_convert_type(((vi & maskk) + bias) | k, jnp.float32)

    a1 = pack(d[:, 0:NG], 0)
    a2 = jnp.full((_TN1, NG), inf)
    a3 = jnp.full((_TN1, NG), inf)
    for k in range(1, GK):
        v = pack(d[:, k * NG:(k + 1) * NG], k)
        t1 = jnp.maximum(a1, v)
        a1 = jnp.minimum(a1, v)
        t2 = jnp.maximum(a2, t1)
        a2 = jnp.minimum(a2, t1)
        a3 = jnp.minimum(a3, t2)

    iota_f = lax.broadcasted_iota(jnp.int32, (_TN1, NG), 1).astype(jnp.float32)
    bigf = jnp.float32(1e9)
    for k in range(_K):
        m = jnp.min(a1, axis=1, keepdims=True)
        cand = jnp.where(a1 == m, iota_f, bigf)
        amc = jnp.min(cand, axis=1, keepdims=True)
        mi = lax.bitcast_convert_type(m, jnp.int32) & 15
        idx_ref[0, k, :] = (mi * NG + amc.astype(jnp.int32))[:, 0]
        hit = iota_f == amc
        a1 = jnp.where(hit, a2, a1)
        a2 = jnp.where(hit, a3, a2)
        a3 = jnp.where(hit, inf, a3)


def _proj_topk(b, feat, posb, post, wenc, benc, wkvq, bkvq):
    return pl.pallas_call(
        _proj_topk_body,
        grid=(_NB1,),
        in_specs=[
            pl.BlockSpec((1, _TN1, _F), lambda i: (b, i, 0)),
            pl.BlockSpec((1, _TN1, 8), lambda i: (b, i, 0)),
            pl.BlockSpec((1, 8, _N), lambda i: (b, 0, 0)),
            pl.BlockSpec((_F, _C), lambda i: (0, 0)),
            pl.BlockSpec((1, _C), lambda i: (0, 0)),
            pl.BlockSpec((_C, 3 * _C), lambda i: (0, 0)),
            pl.BlockSpec((1, 3 * _C), lambda i: (0, 0)),
        ],
        out_specs=[
            pl.BlockSpec((1, _TN1, _C), lambda i: (0, i, 0)),
            pl.BlockSpec((1, _TN1, _TW), lambda i: (0, i, 0)),
            pl.BlockSpec((1, _K, _TN1), lambda i: (0, 0, i)),
        ],
        out_shape=[
            jax.ShapeDtypeStruct((1, _N, _C), jnp.float32),
            jax.ShapeDtypeStruct((1, _N, _TW), jnp.float32),
            jax.ShapeDtypeStruct((1, _K, _N), jnp.int32),
        ],
    )(feat, posb, post, wenc, benc, wkvq, bkvq)


def _sc_gather_body(tab_hbm, idx_hbm, out_hbm, idx_v, rows_v,
                    gsem0, gsem1, osem0, osem1):
    wid = lax.axis_index("s") * 2 + lax.axis_index("c")
    base = wid * _PERW
    gsem = (gsem0, gsem1)
    osem = (osem0, osem1)
    pltpu.sync_copy(idx_hbm.at[pl.ds(base, _PERW)], idx_v)

    def body(g, carry):
        @pl.when(g > 0)
        def _():
            for t in range(2):
                pltpu.make_async_copy(
                    rows_v.at[t], out_hbm.at[pl.ds(base, _CHUNK)],
                    osem[t]).wait()
        cps = []
        for t in range(2):
            c = g * 2 + t
            cps.append(pltpu.async_copy(
                tab_hbm.at[idx_v.at[pl.ds(c * _CHUNK, _CHUNK)]],
                rows_v.at[t], gsem[t]))
        for t in range(2):
            c = g * 2 + t
            cps[t].wait()
            pltpu.async_copy(
                rows_v.at[t], out_hbm.at[pl.ds(base + c * _CHUNK, _CHUNK)],
                osem[t])
        return carry

    lax.fori_loop(0, _NCH // 2, body, 0)
    for t in range(2):
        pltpu.make_async_copy(
            rows_v.at[t], out_hbm.at[pl.ds(base, _CHUNK)], osem[t]).wait()


def _sc_gather(tabf, idxf):
    mesh = plsc.VectorSubcoreMesh(core_axis_name="c", subcore_axis_name="s")
    fn = functools.partial(
        pl.kernel,
        mesh=mesh,
        out_type=jax.ShapeDtypeStruct((_TOTB, _TW), jnp.float32),
        scratch_types=[
            pltpu.VMEM((_PERW,), jnp.int32),
            pltpu.VMEM((2, _CHUNK, _TW), jnp.float32),
            pltpu.SemaphoreType.DMA,
            pltpu.SemaphoreType.DMA,
            pltpu.SemaphoreType.DMA,
            pltpu.SemaphoreType.DMA,
        ],
    )(_sc_gather_body)
    return fn(tabf, idxf)


def _attn_body(g_ref, feat_ref, q_ref, posb_ref,
               wp1_ref, bp1_ref, wp2_ref, bp2_ref,
               wg1_ref, bg1_ref, wg2_ref, bg2_ref,
               wdec_ref, bdec_ref, out_ref):
    g = g_ref[...]
    kt = _K * _TN2
    kn = g[:, :, :_C].reshape(kt, _C)
    vn = g[:, :, _C:2 * _C]
    pn = g[:, :, 2 * _C:2 * _C + _PW].reshape(kt, _PW)

    posb = posb_ref[0]
    pos16 = jnp.concatenate(
        [posb, jnp.zeros((_TN2, _PW - 8), jnp.float32)], axis=1)
    rel = jnp.broadcast_to(pos16[None], (_K, _TN2, _PW)).reshape(kt, _PW) - pn

    h = jnp.maximum(
        jnp.dot(rel, wp1_ref[...], preferred_element_type=jnp.float32) + bp1_ref[...], 0.0)
    delta = jnp.dot(h, wp2_ref[...], preferred_element_type=jnp.float32) + bp2_ref[...]

    qb = q_ref[0]
    qq = jnp.broadcast_to(qb[None], (_K, _TN2, _C)).reshape(kt, _C)
    x = qq - kn + delta
    hg = jnp.maximum(
        jnp.dot(x, wg1_ref[...], preferred_element_type=jnp.float32) + bg1_ref[...], 0.0)
    gam = jnp.dot(hg, wg2_ref[...], preferred_element_type=jnp.float32) + bg2_ref[...]

    gam3 = gam.reshape(_K, _TN2, _C)
    del3 = delta.reshape(_K, _TN2, _C)
    m = gam3[0]
    for k in range(1, _K):
        m = jnp.maximum(m, gam3[k])
    s = jnp.zeros((_TN2, _C), jnp.float32)
    acc = jnp.zeros((_TN2, _C), jnp.float32)
    for k in range(_K):
        e = jnp.exp(gam3[k] - m)
        s = s + e
        acc = acc + e * (vn[k] + del3[k])
    agg = acc / s

    out_ref[0] = feat_ref[0] + (
        jnp.dot(agg, wdec_ref[...], preferred_element_type=jnp.float32) + bdec_ref[...])


def _attn(b, g3, feat, q, posb, wp1p, bp1, wp2, bp2, wg1, bg1, wg2, bg2,
          wdec, bdec):
    wspec = pl.BlockSpec((_C, _C), lambda i: (0, 0))
    bspec = pl.BlockSpec((1, _C), lambda i: (0, 0))
    return pl.pallas_call(
        _attn_body,
        grid=(_NB2,),
        in_specs=[
            pl.BlockSpec((_K, _TN2, _TW), lambda i: (0, i, 0)),
            pl.BlockSpec((1, _TN2, _F), lambda i: (b, i, 0)),
            pl.BlockSpec((1, _TN2, _C), lambda i: (0, i, 0)),
            pl.BlockSpec((1, _TN2, 8), lambda i: (b, i, 0)),
            pl.BlockSpec((_PW, _C), lambda i: (0, 0)),
            bspec, wspec, bspec, wspec, bspec, wspec, bspec,
            pl.BlockSpec((_C, _F), lambda i: (0, 0)),
            pl.BlockSpec((1, _F), lambda i: (0, 0)),
        ],
        out_specs=pl.BlockSpec((1, _TN2, _F), lambda i: (0, i, 0)),
        out_shape=jax.ShapeDtypeStruct((1, _N, _F), jnp.float32),
    )(g3, feat, q, posb, wp1p, bp1, wp2, bp2, wg1, bg1, wg2, bg2, wdec, bdec)


def kernel(features, positions, batch, W_enc, b_enc, W_q, b_q, W_k, b_k,
           W_v, b_v, W_p1, b_p1, W_p2, b_p2, W_g1, b_g1, W_g2, b_g2,
           W_dec, b_dec):
    posb = jnp.concatenate(
        [positions, jnp.zeros((_B, _N, 5), jnp.float32)], axis=2)
    post = jnp.transpose(posb, (0, 2, 1))
    wp1p = jnp.concatenate(
        [W_p1, jnp.zeros((_PW - 3, _C), jnp.float32)], axis=0)
    wkvq = jnp.concatenate([W_k, W_v, W_q], axis=1)
    bkvq = jnp.concatenate([b_k, b_v, b_q]).reshape(1, 3 * _C)

    outs = []
    for b in range(_B):
        q, tab, idx = _proj_topk(
            b, features, posb, post, W_enc, b_enc.reshape(1, _C), wkvq, bkvq)
        gat = _sc_gather(tab.reshape(_N, _TW), idx.reshape(_TOTB))
        g3 = gat.reshape(_K, _N, _TW)
        outs.append(_attn(
            b, g3, features, q, posb, wp1p,
            b_p1.reshape(1, _C), W_p2, b_p2.reshape(1, _C),
            W_g1, b_g1.reshape(1, _C), W_g2, b_g2.reshape(1, _C),
            W_dec, b_dec.reshape(1, _F)))
    out = jnp.concatenate(outs, axis=0)
    return out, positions, batch
